# baseline (speedup 1.0000x reference)
import functools
import math

import jax
import jax.numpy as jnp
from jax import lax
from jax.experimental import pallas as pl
from jax.experimental.pallas import tpu as pltpu
from jax.experimental.pallas import tpu_sc as plsc

_L = 16
_BB = 512
_SUB = 128
_NBUF = 2


@functools.partial(jax.jit, static_argnames=("vv", "d"))
def _table_rowmajor(tt, vv, d):
    info = plsc.get_sparse_core_info()
    nc, ns = info.num_cores, info.num_subcores
    nw = nc * ns
    n_dg = d // 8
    gv = 512
    n_vtile = vv // 128
    tail = vv - n_vtile * 128
    n_grp = n_vtile // (gv // 128)
    n_iter = n_grp // nw
    n_left = n_grp - n_iter * nw
    orow_g = gv * d // 128

    mesh = plsc.VectorSubcoreMesh(core_axis_name="c", subcore_axis_name="s")

    @functools.partial(
        pl.kernel,
        mesh=mesh,
        compiler_params=pltpu.CompilerParams(
            use_tc_tiling_on_sc=True, needs_layout_passes=False
        ),
        out_type=jax.ShapeDtypeStruct((vv * d // 128, 128), jnp.float32),
        scratch_types=[
            pltpu.VMEM((2, d, gv + 1), jnp.float32),
            pltpu.VMEM((2, orow_g, 128), jnp.float32),
            pltpu.SemaphoreType.DMA((2,)),
            pltpu.SemaphoreType.DMA((2,)),
        ],
    )
    def k(tt_hbm, tw_hbm, vbuf, sbuf, gsem, ssem):
        wid = lax.axis_index("s") * nc + lax.axis_index("c")
        lane = lax.iota(jnp.int32, _L)

        def fire_reads(g, b):
            v0 = g * gv
            for dg in range(n_dg):
                pltpu.async_copy(
                    tt_hbm.at[pl.ds(dg * 8, 8), pl.ds(v0, gv)],
                    vbuf.at[b, pl.ds(dg * 8, 8), pl.ds(0, gv)],
                    gsem.at[b],
                )

        def wait_reads(b):
            for dg in range(n_dg):
                pltpu.make_async_copy(
                    tt_hbm.at[pl.ds(0, 8), pl.ds(0, gv)],
                    vbuf.at[b, pl.ds(0, 8), pl.ds(0, gv)],
                    gsem.at[b],
                ).wait()

        def transpose_group(b, nv):
            @plsc.parallel_loop(0, nv, 1, unroll=4)
            def _(v):
                row = lax.shift_right_logical(v, 2)
                c0 = lax.bitwise_and(v, 3) * d
                vs = jnp.full((_L,), v, jnp.int32)
                for jj in range(0, d, _L):
                    x = plsc.load_gather(vbuf.at[b], [jj + lane, vs])
                    sbuf[b, row, pl.ds(c0 + jj, _L)] = x

        for b in range(2):
            fire_reads(b * nw + wid, b)

        def body(i, carry):
            b = lax.rem(i, 2)
            wait_reads(b)

            @pl.when(i >= 2)
            def _():
                pltpu.make_async_copy(
                    sbuf.at[b], tw_hbm.at[pl.ds(0, orow_g)], ssem.at[b]
                ).wait()

            transpose_group(b, gv)
            g = i * nw + wid
            pltpu.async_copy(
                sbuf.at[b], tw_hbm.at[pl.ds(g * orow_g, orow_g)], ssem.at[b]
            )

            @pl.when(i + 2 <= n_iter - 1)
            def _():
                fire_reads((i + 2) * nw + wid, b)

            return carry

        lax.fori_loop(0, n_iter, body, 0)

        for b in range(2):
            pltpu.make_async_copy(
                sbuf.at[b], tw_hbm.at[pl.ds(0, orow_g)], ssem.at[b]
            ).wait()

        @pl.when(wid < n_left)
        def _():
            g = n_iter * nw + wid
            fire_reads(g, 0)
            wait_reads(0)
            transpose_group(0, gv)
            pltpu.sync_copy(
                sbuf.at[0], tw_hbm.at[pl.ds(g * orow_g, orow_g)]
            )


    return k(tt)


@functools.partial(jax.jit, static_argnames=("bsz", "h", "d", "vmain"))
def _emb_lookup(xt, table, aux, bsz, h, d, vmain):
    info = plsc.get_sparse_core_info()
    n_aux = aux.shape[0]
    nc, ns = info.num_cores, info.num_subcores
    nw = nc * ns
    n_bblk = bsz // _BB
    n_hblk = nw // n_bblk
    h_per_w = h // n_hblk
    n_sub = _BB // _SUB
    n_dg = d // 8
    n_bg = bsz // 128
    w_bg = _BB // 128
    scale = math.sqrt(float(d))

    mesh = plsc.VectorSubcoreMesh(core_axis_name="c", subcore_axis_name="s")

    @functools.partial(
        pl.kernel,
        mesh=mesh,
        compiler_params=pltpu.CompilerParams(
            use_tc_tiling_on_sc=False, needs_layout_passes=False
        ),
        out_type=jax.ShapeDtypeStruct((h, n_dg, n_bg, 8, 128), jnp.float32),
        scratch_types=[
            pltpu.VMEM((h_per_w, _BB), jnp.int32),
            pltpu.VMEM((h_per_w, _BB), jnp.int32),
            pltpu.VMEM((_NBUF, _BB, d), jnp.float32),
            pltpu.VMEM((_NBUF, n_dg, w_bg, 8, 129), jnp.float32),
            pltpu.VMEM((n_aux, d), jnp.float32),
            pltpu.SemaphoreType.DMA((_NBUF,)),
            pltpu.SemaphoreType.DMA((_NBUF,)),
        ],
    )
    def k(xt_hbm, table_hbm, aux_hbm, out_hbm,
          idx_v, idx_c, gbuf, tbuf, aux_v, gsem, ssem):
        wid = lax.axis_index("s") * nc + lax.axis_index("c")
        h0 = (wid // n_bblk) * h_per_w
        b0 = (wid % n_bblk) * _BB
        bg0 = (wid % n_bblk) * w_bg
        pltpu.sync_copy(xt_hbm.at[pl.ds(h0, h_per_w), pl.ds(b0, _BB)], idx_v)
        pltpu.sync_copy(aux_hbm, aux_v)

        @plsc.parallel_loop(0, h_per_w * _BB // _L, 1, unroll=4)
        def _(i):
            r = lax.shift_right_logical(i, 5)
            c = lax.bitwise_and(i, 31) * _L
            idx_c[r, pl.ds(c, _L)] = jnp.minimum(
                idx_v[r, pl.ds(c, _L)], vmain - 1
            )

        def fire_gathers(s, b):
            for q in range(n_sub):
                pltpu.async_copy(
                    table_hbm.at[idx_c.at[s, pl.ds(q * _SUB, _SUB)]],
                    gbuf.at[b, pl.ds(q * _SUB, _SUB)],
                    gsem.at[b],
                )

        def wait_gathers(b):
            for q in range(n_sub):
                pltpu.make_async_copy(
                    table_hbm.at[idx_c.at[0, pl.ds(0, _SUB)]],
                    gbuf.at[b, pl.ds(0, _SUB)],
                    gsem.at[b],
                ).wait()

        def store_src(b):
            return tbuf.at[b, :, pl.ds(0, w_bg), :, pl.ds(0, 128)]

        for b in range(_NBUF):
            fire_gathers(b, b)

        lane = lax.iota(jnp.int32, _L)

        def slab_body(s, carry):
            b = lax.rem(s, _NBUF)

            wait_gathers(b)

            for rg in range(_BB // _L):
                iv = idx_v[s, pl.ds(rg * _L, _L)]
                m = iv >= vmain

                @pl.when(jnp.any(m))
                def _():
                    rowc = jnp.clip(iv - vmain, 0, n_aux - 1)
                    for f in range(d):
                        fs = jnp.full((_L,), f, jnp.int32)
                        av = plsc.load_gather(aux_v, [rowc, fs], mask=m)
                        plsc.store_scatter(
                            gbuf.at[b], [rg * _L + lane, fs], av, mask=m
                        )

            @pl.when(s >= _NBUF)
            def _():
                pltpu.make_async_copy(
                    store_src(b),
                    out_hbm.at[0, :, pl.ds(0, w_bg)],
                    ssem.at[b],
                ).wait()

            @plsc.parallel_loop(0, _BB, 1, unroll=4)
            def _(r):
                bg = jnp.full((_L,), lax.shift_right_logical(r, 7), jnp.int32)
                b1 = jnp.full((_L,), lax.bitwise_and(r, 127), jnp.int32)
                for jj in range(0, d, _L):
                    dv = jj + lane
                    v = gbuf[b, r, pl.ds(jj, _L)]
                    plsc.store_scatter(
                        tbuf.at[b],
                        [lax.shift_right_logical(dv, 3), bg,
                         lax.bitwise_and(dv, 7), b1],
                        v * scale,
                    )

            pltpu.async_copy(
                store_src(b),
                out_hbm.at[h0 + s, :, pl.ds(bg0, w_bg)],
                ssem.at[b],
            )

            @pl.when(s < h_per_w - _NBUF)
            def _():
                fire_gathers(s + _NBUF, b)

            return carry

        lax.fori_loop(0, h_per_w, slab_body, 0)

        for b in range(_NBUF):
            pltpu.make_async_copy(
                store_src(b),
                out_hbm.at[0, :, pl.ds(0, w_bg)],
                ssem.at[b],
            ).wait()

    return k(xt, table, aux)


def kernel(x, table):
    bsz, h = x.shape
    v, d = table.shape
    info = plsc.get_sparse_core_info()
    nw = info.num_cores * info.num_subcores
    assert bsz % _BB == 0 and h % (nw // (bsz // _BB)) == 0 and d % 8 == 0
    xt = jnp.transpose(x.astype(jnp.int32), (1, 0))
    tw = _table_rowmajor(jnp.transpose(table, (1, 0)), v, d)
    table_rm = tw.reshape(v, d)
    vmain = (v // 128) * 128
    n_aux = max(v - vmain, 1)
    aux = jax.lax.slice(table, (v - n_aux, 0), (v, d))
    out = _emb_lookup(xt, table_rm, aux, bsz, h, d, vmain)
    out = jnp.transpose(out, (2, 4, 0, 1, 3))
    return out.reshape(bsz, h, d)


# speedup vs baseline: 1.1189x; 1.1189x over previous
import functools
import math

import jax
import jax.numpy as jnp
from jax import lax
from jax.experimental import pallas as pl
from jax.experimental.pallas import tpu as pltpu
from jax.experimental.pallas import tpu_sc as plsc

_L = 16
_BB = 512
_SUB = 128
_NBUF = 2


@functools.partial(jax.jit, static_argnames=("bsz", "h", "d"))
def _emb_lookup(xt, table, bsz, h, d):
    info = plsc.get_sparse_core_info()
    nc, ns = info.num_cores, info.num_subcores
    nw = nc * ns
    n_bblk = bsz // _BB
    n_hblk = nw // n_bblk
    h_per_w = h // n_hblk
    n_sub = _BB // _SUB
    n_dg = d // 8
    n_bg = bsz // 128
    w_bg = _BB // 128
    scale = math.sqrt(float(d))

    mesh = plsc.VectorSubcoreMesh(core_axis_name="c", subcore_axis_name="s")

    @functools.partial(
        pl.kernel,
        mesh=mesh,
        compiler_params=pltpu.CompilerParams(
            use_tc_tiling_on_sc=False, needs_layout_passes=False
        ),
        out_type=jax.ShapeDtypeStruct((h, n_dg, n_bg, 8, 128), jnp.float32),
        scratch_types=[
            pltpu.VMEM((h_per_w, _BB), jnp.int32),
            pltpu.VMEM((_NBUF, _BB, d), jnp.float32),
            pltpu.VMEM((_NBUF, n_dg, w_bg, 8, 129), jnp.float32),
            pltpu.SemaphoreType.DMA((_NBUF,)),
            pltpu.SemaphoreType.DMA((_NBUF,)),
        ],
    )
    def k(xt_hbm, table_hbm, out_hbm, idx_v, gbuf, tbuf, gsem, ssem):
        wid = lax.axis_index("s") * nc + lax.axis_index("c")
        h0 = (wid // n_bblk) * h_per_w
        b0 = (wid % n_bblk) * _BB
        bg0 = (wid % n_bblk) * w_bg
        pltpu.sync_copy(xt_hbm.at[pl.ds(h0, h_per_w), pl.ds(b0, _BB)], idx_v)

        def fire_gathers(s, b):
            for q in range(n_sub):
                pltpu.async_copy(
                    table_hbm.at[idx_v.at[s, pl.ds(q * _SUB, _SUB)]],
                    gbuf.at[b, pl.ds(q * _SUB, _SUB)],
                    gsem.at[b],
                )

        def wait_gathers(b):
            for q in range(n_sub):
                pltpu.make_async_copy(
                    table_hbm.at[idx_v.at[0, pl.ds(0, _SUB)]],
                    gbuf.at[b, pl.ds(0, _SUB)],
                    gsem.at[b],
                ).wait()

        def store_src(b):
            return tbuf.at[b, :, pl.ds(0, w_bg), :, pl.ds(0, 128)]

        for b in range(_NBUF):
            fire_gathers(b, b)

        lane = lax.iota(jnp.int32, _L)

        def slab_body(s, carry):
            b = lax.rem(s, _NBUF)

            wait_gathers(b)

            @pl.when(s >= _NBUF)
            def _():
                pltpu.make_async_copy(
                    store_src(b),
                    out_hbm.at[0, :, pl.ds(0, w_bg)],
                    ssem.at[b],
                ).wait()

            @plsc.parallel_loop(0, _BB, 1, unroll=4)
            def _(r):
                bg = jnp.full((_L,), lax.shift_right_logical(r, 7), jnp.int32)
                b1 = jnp.full((_L,), lax.bitwise_and(r, 127), jnp.int32)
                for jj in range(0, d, _L):
                    dv = jj + lane
                    v = gbuf[b, r, pl.ds(jj, _L)]
                    plsc.store_scatter(
                        tbuf.at[b],
                        [lax.shift_right_logical(dv, 3), bg,
                         lax.bitwise_and(dv, 7), b1],
                        v * scale,
                    )

            pltpu.async_copy(
                store_src(b),
                out_hbm.at[h0 + s, :, pl.ds(bg0, w_bg)],
                ssem.at[b],
            )

            @pl.when(s < h_per_w - _NBUF)
            def _():
                fire_gathers(s + _NBUF, b)

            return carry

        lax.fori_loop(0, h_per_w, slab_body, 0)

        for b in range(_NBUF):
            pltpu.make_async_copy(
                store_src(b),
                out_hbm.at[0, :, pl.ds(0, w_bg)],
                ssem.at[b],
            ).wait()

    return k(xt, table)


def kernel(x, table):
    bsz, h = x.shape
    v, d = table.shape
    info = plsc.get_sparse_core_info()
    nw = info.num_cores * info.num_subcores
    assert bsz % _BB == 0 and h % (nw // (bsz // _BB)) == 0 and d % 8 == 0
    xt = jnp.transpose(x.astype(jnp.int32), (1, 0))
    out = _emb_lookup(xt, table, bsz, h, d)
    out = jnp.transpose(out, (2, 4, 0, 1, 3))
    return out.reshape(bsz, h, d)
